# Initial kernel scaffold; baseline (speedup 1.0000x reference)
import jax
import jax.numpy as jnp
from jax.experimental import pallas as pl


def kernel(node_features, edge_features, pair_indices, edge_kernel, edge_bias, gru_kernel, gru_rkernel, gru_bias):
    raise NotImplementedError("write your pallas kernel here")


# speedup vs baseline: 3.3457x; 3.3457x over previous
import functools

import jax
import jax.numpy as jnp
from jax import lax
from jax.experimental import pallas as pl
from jax.experimental.pallas import tpu as pltpu
from jax.experimental.pallas import tpu_sc as plsc

STEPS = 4

NC = 2
NS = 16
NW = NC * NS
LANE = 128


def _mesh():
    return plsc.VectorSubcoreMesh(core_axis_name="c", subcore_axis_name="s")


_SC_PARAMS = pltpu.CompilerParams(use_tc_tiling_on_sc=False)


def _make_sc_gather(n_nodes, d, e_pad):
    epw = e_pad // NW
    ch = epw // LANE

    @functools.partial(
        pl.kernel,
        out_type=jax.ShapeDtypeStruct((e_pad, d), jnp.float32),
        mesh=_mesh(),
        scratch_types=[
            pltpu.VMEM((ch, LANE), jnp.int32),
            pltpu.VMEM((epw, d), jnp.float32),
            pltpu.SemaphoreType.DMA,
        ],
        compiler_params=_SC_PARAMS,
    )
    def sc_gather(h_hbm, dstr_hbm, out_hbm, idx_v, rows_v, sem):
        wid = lax.axis_index("s") * NC + lax.axis_index("c")
        pltpu.sync_copy(dstr_hbm.at[wid], idx_v)
        descs = [
            pltpu.async_copy(
                h_hbm.at[idx_v.at[j]], rows_v.at[pl.ds(j * LANE, LANE)], sem
            )
            for j in range(ch)
        ]
        for dsc in descs:
            dsc.wait()
        pltpu.sync_copy(rows_v, out_hbm.at[pl.ds(wid * epw, epw)])

    return sc_gather


def _make_sc_scatter(n_pad, d, e_pad):
    epw = e_pad // NW
    ch = epw // LANE
    rpt = n_pad // NS

    @functools.partial(
        pl.kernel,
        out_type=jax.ShapeDtypeStruct((NC, n_pad, d), jnp.float32),
        mesh=_mesh(),
        scratch_types=[
            pltpu.VMEM((ch, LANE), jnp.int32),
            pltpu.VMEM((epw, d), jnp.float32),
            pltpu.VMEM_SHARED((n_pad, d), jnp.float32),
        ],
        compiler_params=_SC_PARAMS,
    )
    def sc_scatter(msg_hbm, srcr_hbm, zero_hbm, out_hbm, idx_v, msg_v, acc_s):
        cid = lax.axis_index("c")
        sid = lax.axis_index("s")
        wid = sid * NC + cid
        pltpu.sync_copy(zero_hbm, acc_s.at[pl.ds(sid * rpt, rpt)])
        plsc.subcore_barrier()
        pltpu.sync_copy(srcr_hbm.at[wid], idx_v)
        pltpu.sync_copy(msg_hbm.at[pl.ds(wid * epw, epw)], msg_v)
        for j in range(ch):
            pltpu.sync_copy(
                msg_v.at[pl.ds(j * LANE, LANE)],
                acc_s.at[idx_v.at[j]],
                add=True,
            )
        plsc.subcore_barrier()
        pltpu.sync_copy(
            acc_s.at[pl.ds(sid * rpt, rpt)],
            out_hbm.at[cid, pl.ds(sid * rpt, rpt)],
        )

    return sc_scatter


def _msg_body(de, d, eft_ref, nbr_ref, wp2_ref, bmat_ref, out_ref):
    ft = eft_ref[...]
    yt = nbr_ref[...].T
    p = jnp.concatenate([yt * ft[k : k + 1, :] for k in range(de)], axis=0)
    mt = jnp.dot(wp2_ref[...], p, preferred_element_type=jnp.float32)
    mt = mt + jnp.dot(bmat_ref[...], yt, preferred_element_type=jnp.float32)
    out_ref[...] = mt.T


def _make_tc_msg(de, d, e_pad, be):
    grid = (e_pad // be,)
    return pl.pallas_call(
        functools.partial(_msg_body, de, d),
        grid=grid,
        in_specs=[
            pl.BlockSpec((de, be), lambda i: (0, i)),
            pl.BlockSpec((be, d), lambda i: (i, 0)),
            pl.BlockSpec((d, de * d), lambda i: (0, 0)),
            pl.BlockSpec((d, d), lambda i: (0, 0)),
        ],
        out_specs=pl.BlockSpec((be, d), lambda i: (i, 0)),
        out_shape=jax.ShapeDtypeStruct((e_pad, d), jnp.float32),
    )


def _gru_body(agg_ref, h_ref, kz_ref, kr_ref, kh_ref, rkz_ref, rkr_ref,
              rkh_ref, bz_ref, br_ref, b0h_ref, b1h_ref, out_ref):
    a = agg_ref[0] + agg_ref[1]
    h = h_ref[...]
    f32 = jnp.float32
    z = jax.nn.sigmoid(
        jnp.dot(a, kz_ref[...], preferred_element_type=f32)
        + jnp.dot(h, rkz_ref[...], preferred_element_type=f32)
        + bz_ref[...]
    )
    r = jax.nn.sigmoid(
        jnp.dot(a, kr_ref[...], preferred_element_type=f32)
        + jnp.dot(h, rkr_ref[...], preferred_element_type=f32)
        + br_ref[...]
    )
    hh = jnp.tanh(
        jnp.dot(a, kh_ref[...], preferred_element_type=f32)
        + b0h_ref[...]
        + r * (jnp.dot(h, rkh_ref[...], preferred_element_type=f32)
               + b1h_ref[...])
    )
    out_ref[...] = z * h + (1.0 - z) * hh


def _make_tc_gru(n, n_pad, d, bn):
    grid = (n // bn,)
    wspec = pl.BlockSpec((d, d), lambda i: (0, 0))
    bspec = pl.BlockSpec((1, d), lambda i: (0, 0))
    return pl.pallas_call(
        _gru_body,
        grid=grid,
        in_specs=[
            pl.BlockSpec((NC, bn, d), lambda i: (0, i, 0)),
            pl.BlockSpec((bn, d), lambda i: (i, 0)),
            wspec, wspec, wspec, wspec, wspec, wspec,
            bspec, bspec, bspec, bspec,
        ],
        out_specs=pl.BlockSpec((bn, d), lambda i: (i, 0)),
        out_shape=jax.ShapeDtypeStruct((n, d), jnp.float32),
    )


def kernel(node_features, edge_features, pair_indices, edge_kernel,
           edge_bias, gru_kernel, gru_rkernel, gru_bias):
    n, nfc_in = node_features.shape
    e, de = edge_features.shape
    d = gru_kernel.shape[0]
    assert edge_kernel.shape == (de, d * d)

    h = node_features
    if nfc_in < d:
        h = jnp.pad(h, ((0, 0), (0, d - nfc_in)))

    quant = NW * LANE
    e_pad = ((e + quant - 1) // quant) * quant
    ch = e_pad // (NW * LANE)
    rpt = -(-(n + 1) // NS)
    rpt = ((rpt + 7) // 8) * 8
    n_pad = rpt * NS

    src = pair_indices[:, 0]
    dst = pair_indices[:, 1]
    pad_e = e_pad - e
    dst_r = jnp.concatenate(
        [dst, jnp.zeros((pad_e,), jnp.int32)]).reshape(NW, ch, LANE)
    src_r = jnp.concatenate(
        [src, jnp.full((pad_e,), n, jnp.int32)]).reshape(NW, ch, LANE)
    eft = jnp.concatenate(
        [edge_features, jnp.zeros((pad_e, de), jnp.float32)]).T
    zero_blk = jnp.zeros((n_pad // NS, d), jnp.float32)

    wp2 = edge_kernel.reshape(de, d, d).transpose(1, 0, 2).reshape(d, de * d)
    bmat = edge_bias.reshape(d, d)
    kz, kr, kh = (gru_kernel[:, :d], gru_kernel[:, d:2 * d],
                  gru_kernel[:, 2 * d:])
    rkz, rkr, rkh = (gru_rkernel[:, :d], gru_rkernel[:, d:2 * d],
                     gru_rkernel[:, 2 * d:])
    bz = (gru_bias[0, :d] + gru_bias[1, :d]).reshape(1, d)
    br = (gru_bias[0, d:2 * d] + gru_bias[1, d:2 * d]).reshape(1, d)
    b0h = gru_bias[0, 2 * d:].reshape(1, d)
    b1h = gru_bias[1, 2 * d:].reshape(1, d)

    sc_gather = _make_sc_gather(n, d, e_pad)
    sc_scatter = _make_sc_scatter(n_pad, d, e_pad)
    tc_msg = _make_tc_msg(de, d, e_pad, 1024)
    tc_gru = _make_tc_gru(n, n_pad, d, 1000)

    for _ in range(STEPS):
        nbr = sc_gather(h, dst_r)
        msg = tc_msg(eft, nbr, wp2, bmat)
        aggp = sc_scatter(msg, src_r, zero_blk)
        h = tc_gru(aggp, h, kz, kr, kh, rkz, rkr, rkh, bz, br, b0h, b1h)
    return h


# speedup vs baseline: 5.9070x; 1.7655x over previous
import functools

import jax
import jax.numpy as jnp
from jax import lax
from jax.experimental import pallas as pl
from jax.experimental.pallas import tpu as pltpu
from jax.experimental.pallas import tpu_sc as plsc

STEPS = 4

NC = 2
NS = 16
NW = NC * NS
LANE = 128


def _mesh():
    return plsc.VectorSubcoreMesh(core_axis_name="c", subcore_axis_name="s")


_SC_PARAMS = pltpu.CompilerParams(use_tc_tiling_on_sc=False)


def _make_sc_gather(n_nodes, d, e_pad):
    epw = e_pad // NW
    ch = epw // LANE

    @functools.partial(
        pl.kernel,
        out_type=jax.ShapeDtypeStruct((e_pad, d), jnp.float32),
        mesh=_mesh(),
        scratch_types=[
            pltpu.VMEM((ch, LANE), jnp.int32),
            pltpu.VMEM((epw, d), jnp.float32),
            pltpu.SemaphoreType.DMA,
        ],
        compiler_params=_SC_PARAMS,
    )
    def sc_gather(h_hbm, dstr_hbm, out_hbm, idx_v, rows_v, sem):
        wid = lax.axis_index("s") * NC + lax.axis_index("c")
        pltpu.sync_copy(dstr_hbm.at[wid], idx_v)
        descs = [
            pltpu.async_copy(
                h_hbm.at[idx_v.at[j]], rows_v.at[pl.ds(j * LANE, LANE)], sem
            )
            for j in range(ch)
        ]
        for dsc in descs:
            dsc.wait()
        pltpu.sync_copy(rows_v, out_hbm.at[pl.ds(wid * epw, epw)])

    return sc_gather


def _make_sc_scatter(n_pad, d, e_pad):
    epw = e_pad // NW
    ch = epw // LANE
    rpt = n_pad // NS

    @functools.partial(
        pl.kernel,
        out_type=jax.ShapeDtypeStruct((NC, n_pad, d), jnp.float32),
        mesh=_mesh(),
        scratch_types=[
            pltpu.VMEM((ch, LANE), jnp.int32),
            pltpu.VMEM((epw, d), jnp.float32),
            pltpu.VMEM_SHARED((n_pad, d), jnp.float32),
        ],
        compiler_params=_SC_PARAMS,
    )
    def sc_scatter(msg_hbm, srcr_hbm, zero_hbm, out_hbm, idx_v, msg_v, acc_s):
        cid = lax.axis_index("c")
        sid = lax.axis_index("s")
        wid = sid * NC + cid
        pltpu.sync_copy(zero_hbm, acc_s.at[pl.ds(sid * rpt, rpt)])
        plsc.subcore_barrier()
        pltpu.sync_copy(srcr_hbm.at[wid], idx_v)
        pltpu.sync_copy(msg_hbm.at[pl.ds(wid * epw, epw)], msg_v)
        for j in range(ch):
            pltpu.sync_copy(
                msg_v.at[pl.ds(j * LANE, LANE)],
                acc_s.at[idx_v.at[j]],
                add=True,
            )
        plsc.subcore_barrier()
        pltpu.sync_copy(
            acc_s.at[pl.ds(sid * rpt, rpt)],
            out_hbm.at[cid, pl.ds(sid * rpt, rpt)],
        )

    return sc_scatter


def _msg_body(de, d, fts_ref, nbr4_ref, wp2_ref, bmat_ref, out_ref):
    ytp = nbr4_ref[...].T
    wp2 = wp2_ref[...]
    bmat = bmat_ref[...]
    parts = []
    for a in range(4):
        yt = ytp[d * a : d * (a + 1), :]
        ft = fts_ref[de * a : de * (a + 1), :]
        p = jnp.concatenate(
            [yt * ft[k : k + 1, :] for k in range(de)], axis=0)
        mt = jnp.dot(wp2, p, preferred_element_type=jnp.float32)
        mt = mt + jnp.dot(bmat, yt, preferred_element_type=jnp.float32)
        parts.append(mt)
    out_ref[...] = jnp.concatenate(parts, axis=0).T


def _make_tc_msg(de, d, e_pad, bg):
    g4 = e_pad // 4
    grid = (g4 // bg,)
    return pl.pallas_call(
        functools.partial(_msg_body, de, d),
        grid=grid,
        in_specs=[
            pl.BlockSpec((4 * de, bg), lambda i: (0, i)),
            pl.BlockSpec((bg, 4 * d), lambda i: (i, 0)),
            pl.BlockSpec((d, de * d), lambda i: (0, 0)),
            pl.BlockSpec((d, d), lambda i: (0, 0)),
        ],
        out_specs=pl.BlockSpec((bg, 4 * d), lambda i: (i, 0)),
        out_shape=jax.ShapeDtypeStruct((g4, 4 * d), jnp.float32),
    )


def _gru_body(d, agg4_ref, h4_ref, wbt_ref, bz_ref, br_ref, b0h_ref,
              b1h_ref, out_ref):
    at = (agg4_ref[0] + agg4_ref[1]).T
    ht = h4_ref[...].T
    wbt = wbt_ref[...]
    bz, br, b0h, b1h = bz_ref[...], br_ref[...], b0h_ref[...], b1h_ref[...]
    parts = []
    for a in range(4):
        aa = at[d * a : d * (a + 1), :]
        hh_in = ht[d * a : d * (a + 1), :]
        x = jnp.concatenate([aa, hh_in], axis=0)
        m = jnp.dot(wbt, x, preferred_element_type=jnp.float32)
        z = jax.nn.sigmoid(m[0 : d, :] + bz)
        r = jax.nn.sigmoid(m[d : 2 * d, :] + br)
        cand = jnp.tanh(m[2 * d : 3 * d, :] + b0h
                        + r * (m[3 * d : 4 * d, :] + b1h))
        parts.append(z * hh_in + (1.0 - z) * cand)
    out_ref[...] = jnp.concatenate(parts, axis=0).T


def _make_tc_gru(n_pad, d, bg):
    g4 = n_pad // 4
    grid = (g4 // bg,)
    return pl.pallas_call(
        functools.partial(_gru_body, d),
        grid=grid,
        in_specs=[
            pl.BlockSpec((NC, bg, 4 * d), lambda i: (0, i, 0)),
            pl.BlockSpec((bg, 4 * d), lambda i: (i, 0)),
            pl.BlockSpec((4 * d, 2 * d), lambda i: (0, 0)),
            pl.BlockSpec((d, 1), lambda i: (0, 0)),
            pl.BlockSpec((d, 1), lambda i: (0, 0)),
            pl.BlockSpec((d, 1), lambda i: (0, 0)),
            pl.BlockSpec((d, 1), lambda i: (0, 0)),
        ],
        out_specs=pl.BlockSpec((bg, 4 * d), lambda i: (i, 0)),
        out_shape=jax.ShapeDtypeStruct((g4, 4 * d), jnp.float32),
    )


def kernel(node_features, edge_features, pair_indices, edge_kernel,
           edge_bias, gru_kernel, gru_rkernel, gru_bias):
    n, nfc_in = node_features.shape
    e, de = edge_features.shape
    d = gru_kernel.shape[0]
    assert edge_kernel.shape == (de, d * d)
    assert n % 4 == 0 and d == 32 and de == 16

    h = node_features
    if nfc_in < d:
        h = jnp.pad(h, ((0, 0), (0, d - nfc_in)))

    quant = NW * LANE
    e_pad = ((e + quant - 1) // quant) * quant
    ch = e_pad // (NW * LANE)
    rpt = -(-(n + 1) // NS)
    rpt = ((rpt + 7) // 8) * 8
    n_pad = rpt * NS

    src = pair_indices[:, 0]
    dst = pair_indices[:, 1]
    pad_e = e_pad - e
    dst_r = jnp.concatenate(
        [dst, jnp.zeros((pad_e,), jnp.int32)]).reshape(NW, ch, LANE)
    src_r = jnp.concatenate(
        [src, jnp.full((pad_e,), n, jnp.int32)]).reshape(NW, ch, LANE)
    ef_pad = jnp.concatenate(
        [edge_features, jnp.zeros((pad_e, de), jnp.float32)])
    fts = ef_pad.reshape(e_pad // 4, 4, de).transpose(1, 2, 0).reshape(
        4 * de, e_pad // 4)
    zero_blk = jnp.zeros((n_pad // NS, d), jnp.float32)

    wp2 = edge_kernel.reshape(de, d, d).transpose(1, 0, 2).reshape(d, de * d)
    bmat = edge_bias.reshape(d, d)
    kz, kr, kh = (gru_kernel[:, :d], gru_kernel[:, d:2 * d],
                  gru_kernel[:, 2 * d:])
    rkz, rkr, rkh = (gru_rkernel[:, :d], gru_rkernel[:, d:2 * d],
                     gru_rkernel[:, 2 * d:])
    zer = jnp.zeros((d, d), jnp.float32)
    wbig = jnp.concatenate([
        jnp.concatenate([kz, kr, kh, zer], axis=1),
        jnp.concatenate([rkz, rkr, zer, rkh], axis=1),
    ], axis=0)
    wbt = wbig.T
    bz = (gru_bias[0, :d] + gru_bias[1, :d]).reshape(d, 1)
    br = (gru_bias[0, d:2 * d] + gru_bias[1, d:2 * d]).reshape(d, 1)
    b0h = gru_bias[0, 2 * d:].reshape(d, 1)
    b1h = gru_bias[1, 2 * d:].reshape(d, 1)

    g4 = n_pad // 4
    bgn = g4
    for cand in range(632, 7, -8):
        if g4 % cand == 0:
            bgn = cand
            break

    sc_gather = _make_sc_gather(n_pad, d, e_pad)
    sc_scatter = _make_sc_scatter(n_pad, d, e_pad)
    tc_msg = _make_tc_msg(de, d, e_pad, 512)
    tc_gru = _make_tc_gru(n_pad, d, bgn)

    h4 = jnp.pad(h, ((0, n_pad - n), (0, 0))).reshape(g4, 4 * d)
    for _ in range(STEPS):
        nbr = sc_gather(h4.reshape(n_pad, d), dst_r)
        msg4 = tc_msg(fts, nbr.reshape(e_pad // 4, 4 * d), wp2, bmat)
        aggp = sc_scatter(msg4.reshape(e_pad, d), src_r, zero_blk)
        agg4 = aggp.reshape(NC, g4, 4 * d)
        h4 = tc_gru(agg4, h4, wbt, bz, br, b0h, b1h)
    return h4.reshape(n_pad, d)[:n]


# speedup vs baseline: 6.8416x; 1.1582x over previous
import functools

import jax
import jax.numpy as jnp
from jax import lax
from jax.experimental import pallas as pl
from jax.experimental.pallas import tpu as pltpu
from jax.experimental.pallas import tpu_sc as plsc

STEPS = 4

NC = 2
NS = 16
NW = NC * NS
LANE = 128


def _mesh():
    return plsc.VectorSubcoreMesh(core_axis_name="c", subcore_axis_name="s")


_SC_PARAMS = pltpu.CompilerParams(use_tc_tiling_on_sc=False)


def _make_sc_gather(n_nodes, d, e_pad):
    epw = e_pad // NW
    ch = epw // LANE

    @functools.partial(
        pl.kernel,
        out_type=jax.ShapeDtypeStruct((e_pad, d), jnp.float32),
        mesh=_mesh(),
        scratch_types=[
            pltpu.VMEM((ch, LANE), jnp.int32),
            pltpu.VMEM((epw, d), jnp.float32),
            pltpu.SemaphoreType.DMA,
        ],
        compiler_params=_SC_PARAMS,
    )
    def sc_gather(h_hbm, dstr_hbm, out_hbm, idx_v, rows_v, sem):
        wid = lax.axis_index("s") * NC + lax.axis_index("c")
        pltpu.sync_copy(dstr_hbm.at[wid], idx_v)
        descs = [
            pltpu.async_copy(
                h_hbm.at[idx_v.at[j]], rows_v.at[pl.ds(j * LANE, LANE)], sem
            )
            for j in range(ch)
        ]
        for dsc in descs:
            dsc.wait()
        pltpu.sync_copy(rows_v, out_hbm.at[pl.ds(wid * epw, epw)])

    return sc_gather


def _make_sc_scatter(n_pad, d, e_pad):
    epw = e_pad // NW
    ch = epw // LANE
    rpt = n_pad // NS

    @functools.partial(
        pl.kernel,
        out_type=jax.ShapeDtypeStruct((NC, n_pad, d), jnp.float32),
        mesh=_mesh(),
        scratch_types=[
            pltpu.VMEM((ch, LANE), jnp.int32),
            pltpu.VMEM((epw, d), jnp.float32),
            pltpu.VMEM_SHARED((n_pad, d), jnp.float32),
        ],
        compiler_params=_SC_PARAMS,
    )
    def sc_scatter(msg_hbm, srcr_hbm, zero_hbm, out_hbm, idx_v, msg_v, acc_s):
        cid = lax.axis_index("c")
        sid = lax.axis_index("s")
        wid = sid * NC + cid
        pltpu.sync_copy(zero_hbm, acc_s.at[pl.ds(sid * rpt, rpt)])
        plsc.subcore_barrier()
        pltpu.sync_copy(srcr_hbm.at[wid], idx_v)
        pltpu.sync_copy(msg_hbm.at[pl.ds(wid * epw, epw)], msg_v)
        for j in range(ch):
            pltpu.sync_copy(
                msg_v.at[pl.ds(j * LANE, LANE)],
                acc_s.at[idx_v.at[j]],
                add=True,
            )
        plsc.subcore_barrier()
        pltpu.sync_copy(
            acc_s.at[pl.ds(sid * rpt, rpt)],
            out_hbm.at[cid, pl.ds(sid * rpt, rpt)],
        )

    return sc_scatter


def _msg_body(de, d, ft0_ref, ft1_ref, ft2_ref, ft3_ref, nbr4_ref, wpa_ref,
              out_ref):
    ytp = nbr4_ref[...].T
    wpa = wpa_ref[...]
    fts = (ft0_ref, ft1_ref, ft2_ref, ft3_ref)
    parts = []
    for a in range(4):
        yt = ytp[d * a : d * (a + 1), :]
        ft = fts[a][...]
        p = jnp.concatenate(
            [yt * ft[k : k + 1, :] for k in range(de)] + [yt], axis=0)
        mt = jnp.dot(wpa, p, preferred_element_type=jnp.float32)
        parts.append(mt)
    out_ref[...] = jnp.concatenate(parts, axis=0).T


def _make_tc_msg(de, d, e_pad, bg):
    g4 = e_pad // 4
    grid = (g4 // bg,)
    nb = g4 // bg

    def ftmap(a):
        return lambda i: (0, a * nb + i)

    return pl.pallas_call(
        functools.partial(_msg_body, de, d),
        grid=grid,
        in_specs=[
            pl.BlockSpec((de, bg), ftmap(0)),
            pl.BlockSpec((de, bg), ftmap(1)),
            pl.BlockSpec((de, bg), ftmap(2)),
            pl.BlockSpec((de, bg), ftmap(3)),
            pl.BlockSpec((bg, 4 * d), lambda i: (i, 0)),
            pl.BlockSpec((d, de * d + d), lambda i: (0, 0)),
        ],
        out_specs=pl.BlockSpec((bg, 4 * d), lambda i: (i, 0)),
        out_shape=jax.ShapeDtypeStruct((g4, 4 * d), jnp.float32),
    )


def _gru_body(d, agg4_ref, h4_ref, wbt_ref, bz_ref, br_ref, b0h_ref,
              b1h_ref, out_ref):
    at = (agg4_ref[0] + agg4_ref[1]).T
    ht = h4_ref[...].T
    wbt = wbt_ref[...]
    bz, br, b0h, b1h = bz_ref[...], br_ref[...], b0h_ref[...], b1h_ref[...]
    parts = []
    for a in range(4):
        aa = at[d * a : d * (a + 1), :]
        hh_in = ht[d * a : d * (a + 1), :]
        x = jnp.concatenate([aa, hh_in], axis=0)
        m = jnp.dot(wbt, x, preferred_element_type=jnp.float32)
        z = jax.nn.sigmoid(m[0 : d, :] + bz)
        r = jax.nn.sigmoid(m[d : 2 * d, :] + br)
        cand = jnp.tanh(m[2 * d : 3 * d, :] + b0h
                        + r * (m[3 * d : 4 * d, :] + b1h))
        parts.append(z * hh_in + (1.0 - z) * cand)
    out_ref[...] = jnp.concatenate(parts, axis=0).T


def _make_tc_gru(n_pad, d, bg):
    g4 = n_pad // 4
    grid = (g4 // bg,)
    return pl.pallas_call(
        functools.partial(_gru_body, d),
        grid=grid,
        in_specs=[
            pl.BlockSpec((NC, bg, 4 * d), lambda i: (0, i, 0)),
            pl.BlockSpec((bg, 4 * d), lambda i: (i, 0)),
            pl.BlockSpec((4 * d, 2 * d), lambda i: (0, 0)),
            pl.BlockSpec((d, 1), lambda i: (0, 0)),
            pl.BlockSpec((d, 1), lambda i: (0, 0)),
            pl.BlockSpec((d, 1), lambda i: (0, 0)),
            pl.BlockSpec((d, 1), lambda i: (0, 0)),
        ],
        out_specs=pl.BlockSpec((bg, 4 * d), lambda i: (i, 0)),
        out_shape=jax.ShapeDtypeStruct((g4, 4 * d), jnp.float32),
    )


def kernel(node_features, edge_features, pair_indices, edge_kernel,
           edge_bias, gru_kernel, gru_rkernel, gru_bias):
    n, nfc_in = node_features.shape
    e, de = edge_features.shape
    d = gru_kernel.shape[0]
    assert edge_kernel.shape == (de, d * d)
    assert n % 4 == 0 and d == 32 and de == 16

    h = node_features
    if nfc_in < d:
        h = jnp.pad(h, ((0, 0), (0, d - nfc_in)))

    quant = NW * LANE
    e_pad = ((e + quant - 1) // quant) * quant
    ch = e_pad // (NW * LANE)
    rpt = -(-(n + 1) // NS)
    rpt = ((rpt + 7) // 8) * 8
    n_pad = rpt * NS

    src = pair_indices[:, 0]
    dst = pair_indices[:, 1]
    pad_e = e_pad - e
    g4e = e_pad // 4
    pp = jnp.arange(e_pad, dtype=jnp.int32)
    sigma = (pp % 4) * g4e + pp // 4
    dst_p = jnp.concatenate([dst, jnp.zeros((pad_e,), jnp.int32)])[sigma]
    src_p = jnp.concatenate([src, jnp.full((pad_e,), n, jnp.int32)])[sigma]
    dst_r = dst_p.reshape(NW, ch, LANE)
    src_r = src_p.reshape(NW, ch, LANE)
    eftT = jnp.pad(edge_features, ((0, pad_e), (0, 0))).T
    zero_blk = jnp.zeros((n_pad // NS, d), jnp.float32)

    wp2 = edge_kernel.reshape(de, d, d).transpose(1, 0, 2).reshape(d, de * d)
    wpa = jnp.concatenate([wp2, edge_bias.reshape(d, d)], axis=1)
    kz, kr, kh = (gru_kernel[:, :d], gru_kernel[:, d:2 * d],
                  gru_kernel[:, 2 * d:])
    rkz, rkr, rkh = (gru_rkernel[:, :d], gru_rkernel[:, d:2 * d],
                     gru_rkernel[:, 2 * d:])
    zer = jnp.zeros((d, d), jnp.float32)
    wbig = jnp.concatenate([
        jnp.concatenate([kz, kr, kh, zer], axis=1),
        jnp.concatenate([rkz, rkr, zer, rkh], axis=1),
    ], axis=0)
    wbt = wbig.T
    bz = (gru_bias[0, :d] + gru_bias[1, :d]).reshape(d, 1)
    br = (gru_bias[0, d:2 * d] + gru_bias[1, d:2 * d]).reshape(d, 1)
    b0h = gru_bias[0, 2 * d:].reshape(d, 1)
    b1h = gru_bias[1, 2 * d:].reshape(d, 1)

    g4 = n_pad // 4
    bgn = g4
    for cand in range(632, 7, -8):
        if g4 % cand == 0:
            bgn = cand
            break

    sc_gather = _make_sc_gather(n_pad, d, e_pad)
    sc_scatter = _make_sc_scatter(n_pad, d, e_pad)
    tc_msg = _make_tc_msg(de, d, e_pad, 512)
    tc_gru = _make_tc_gru(n_pad, d, bgn)

    h4 = jnp.pad(h, ((0, n_pad - n), (0, 0))).reshape(g4, 4 * d)
    for _ in range(STEPS):
        nbr = sc_gather(h4.reshape(n_pad, d), dst_r)
        msg4 = tc_msg(eftT, eftT, eftT, eftT,
                      nbr.reshape(e_pad // 4, 4 * d), wpa)
        aggp = sc_scatter(msg4.reshape(e_pad, d), src_r, zero_blk)
        agg4 = aggp.reshape(NC, g4, 4 * d)
        h4 = tc_gru(agg4, h4, wbt, bz, br, b0h, b1h)
    return h4.reshape(n_pad, d)[:n]


# speedup vs baseline: 8.1523x; 1.1916x over previous
import functools

import jax
import jax.numpy as jnp
from jax import lax
from jax.experimental import pallas as pl
from jax.experimental.pallas import tpu as pltpu
from jax.experimental.pallas import tpu_sc as plsc

STEPS = 4

NC = 2
NS = 16
NW = NC * NS
LANE = 128


def _mesh():
    return plsc.VectorSubcoreMesh(core_axis_name="c", subcore_axis_name="s")


_SC_PARAMS = pltpu.CompilerParams(use_tc_tiling_on_sc=False)


def _make_sc_gather(n_nodes, d, e_pad):
    epw = e_pad // NW
    ch = epw // LANE

    rpt = n_nodes // NS

    @functools.partial(
        pl.kernel,
        out_type=jax.ShapeDtypeStruct((e_pad, d), jnp.float32),
        mesh=_mesh(),
        scratch_types=[
            pltpu.VMEM((ch, LANE), jnp.int32),
            pltpu.VMEM((epw, d), jnp.float32),
            pltpu.VMEM_SHARED((n_nodes, d), jnp.float32),
            pltpu.SemaphoreType.DMA,
        ],
        compiler_params=_SC_PARAMS,
    )
    def sc_gather(h_hbm, dstr_hbm, out_hbm, idx_v, rows_v, h_s, sem):
        cid = lax.axis_index("c")
        sid = lax.axis_index("s")
        wid = sid * NC + cid
        pltpu.sync_copy(h_hbm.at[pl.ds(sid * rpt, rpt)],
                        h_s.at[pl.ds(sid * rpt, rpt)])
        pltpu.sync_copy(dstr_hbm.at[wid], idx_v)
        plsc.subcore_barrier()
        descs = [
            pltpu.async_copy(
                h_s.at[idx_v.at[j]], rows_v.at[pl.ds(j * LANE, LANE)], sem
            )
            for j in range(ch)
        ]
        for dsc in descs:
            dsc.wait()
        pltpu.sync_copy(rows_v, out_hbm.at[pl.ds(wid * epw, epw)])

    return sc_gather


def _make_sc_scatter(n_pad, d, e_pad):
    epw = e_pad // NW
    ch = epw // LANE
    rpt = n_pad // NS

    @functools.partial(
        pl.kernel,
        out_type=jax.ShapeDtypeStruct((NC, n_pad, d), jnp.float32),
        mesh=_mesh(),
        scratch_types=[
            pltpu.VMEM((ch, LANE), jnp.int32),
            pltpu.VMEM((epw, d), jnp.float32),
            pltpu.VMEM_SHARED((n_pad, d), jnp.float32),
        ],
        compiler_params=_SC_PARAMS,
    )
    def sc_scatter(msg_hbm, srcr_hbm, zero_hbm, out_hbm, idx_v, msg_v, acc_s):
        cid = lax.axis_index("c")
        sid = lax.axis_index("s")
        wid = sid * NC + cid
        pltpu.sync_copy(zero_hbm, acc_s.at[pl.ds(sid * rpt, rpt)])
        plsc.subcore_barrier()
        pltpu.sync_copy(srcr_hbm.at[wid], idx_v)
        pltpu.sync_copy(msg_hbm.at[pl.ds(wid * epw, epw)], msg_v)
        for j in range(ch):
            pltpu.sync_copy(
                msg_v.at[pl.ds(j * LANE, LANE)],
                acc_s.at[idx_v.at[j]],
                add=True,
            )
        plsc.subcore_barrier()
        pltpu.sync_copy(
            acc_s.at[pl.ds(sid * rpt, rpt)],
            out_hbm.at[cid, pl.ds(sid * rpt, rpt)],
        )

    return sc_scatter


def _msg_body(de, d, ft0_ref, ft1_ref, ft2_ref, ft3_ref, nbr4_ref, wpa_ref,
              out_ref):
    ytp = nbr4_ref[...].T
    wpa = wpa_ref[...]
    fts = (ft0_ref, ft1_ref, ft2_ref, ft3_ref)
    parts = []
    for a in range(4):
        yt = ytp[d * a : d * (a + 1), :]
        ft = fts[a][...]
        p = jnp.concatenate(
            [yt * ft[k : k + 1, :] for k in range(de)] + [yt], axis=0)
        mt = jnp.dot(wpa, p, preferred_element_type=jnp.float32)
        parts.append(mt)
    out_ref[...] = jnp.concatenate(parts, axis=0).T


def _make_tc_msg(de, d, e_pad, bg):
    g4 = e_pad // 4
    grid = (g4 // bg,)
    nb = g4 // bg

    def ftmap(a):
        return lambda i: (0, a * nb + i)

    return pl.pallas_call(
        functools.partial(_msg_body, de, d),
        grid=grid,
        in_specs=[
            pl.BlockSpec((de, bg), ftmap(0)),
            pl.BlockSpec((de, bg), ftmap(1)),
            pl.BlockSpec((de, bg), ftmap(2)),
            pl.BlockSpec((de, bg), ftmap(3)),
            pl.BlockSpec((bg, 4 * d), lambda i: (i, 0)),
            pl.BlockSpec((d, de * d + d), lambda i: (0, 0)),
        ],
        out_specs=pl.BlockSpec((bg, 4 * d), lambda i: (i, 0)),
        out_shape=jax.ShapeDtypeStruct((g4, 4 * d), jnp.float32),
    )


def _gru_body(d, agg4_ref, h4_ref, wbt_ref, bz_ref, br_ref, b0h_ref,
              b1h_ref, out_ref):
    at = (agg4_ref[0] + agg4_ref[1]).T
    ht = h4_ref[...].T
    wbt = wbt_ref[...]
    bz, br, b0h, b1h = bz_ref[...], br_ref[...], b0h_ref[...], b1h_ref[...]
    parts = []
    for a in range(4):
        aa = at[d * a : d * (a + 1), :]
        hh_in = ht[d * a : d * (a + 1), :]
        x = jnp.concatenate([aa, hh_in], axis=0)
        m = jnp.dot(wbt, x, preferred_element_type=jnp.float32)
        z = jax.nn.sigmoid(m[0 : d, :] + bz)
        r = jax.nn.sigmoid(m[d : 2 * d, :] + br)
        cand = jnp.tanh(m[2 * d : 3 * d, :] + b0h
                        + r * (m[3 * d : 4 * d, :] + b1h))
        parts.append(z * hh_in + (1.0 - z) * cand)
    out_ref[...] = jnp.concatenate(parts, axis=0).T


def _make_tc_gru(n_pad, d, bg):
    g4 = n_pad // 4
    grid = (g4 // bg,)
    return pl.pallas_call(
        functools.partial(_gru_body, d),
        grid=grid,
        in_specs=[
            pl.BlockSpec((NC, bg, 4 * d), lambda i: (0, i, 0)),
            pl.BlockSpec((bg, 4 * d), lambda i: (i, 0)),
            pl.BlockSpec((4 * d, 2 * d), lambda i: (0, 0)),
            pl.BlockSpec((d, 1), lambda i: (0, 0)),
            pl.BlockSpec((d, 1), lambda i: (0, 0)),
            pl.BlockSpec((d, 1), lambda i: (0, 0)),
            pl.BlockSpec((d, 1), lambda i: (0, 0)),
        ],
        out_specs=pl.BlockSpec((bg, 4 * d), lambda i: (i, 0)),
        out_shape=jax.ShapeDtypeStruct((g4, 4 * d), jnp.float32),
    )


def kernel(node_features, edge_features, pair_indices, edge_kernel,
           edge_bias, gru_kernel, gru_rkernel, gru_bias):
    n, nfc_in = node_features.shape
    e, de = edge_features.shape
    d = gru_kernel.shape[0]
    assert edge_kernel.shape == (de, d * d)
    assert n % 4 == 0 and d == 32 and de == 16

    h = node_features
    if nfc_in < d:
        h = jnp.pad(h, ((0, 0), (0, d - nfc_in)))

    quant = NW * LANE
    e_pad = ((e + quant - 1) // quant) * quant
    ch = e_pad // (NW * LANE)
    rpt = -(-(n + 1) // NS)
    rpt = ((rpt + 7) // 8) * 8
    n_pad = rpt * NS

    src = pair_indices[:, 0]
    dst = pair_indices[:, 1]
    pad_e = e_pad - e
    g4e = e_pad // 4
    dst_p = jnp.concatenate(
        [dst, jnp.zeros((pad_e,), jnp.int32)]).reshape(4, g4e).T.reshape(-1)
    src_p = jnp.concatenate(
        [src, jnp.full((pad_e,), n, jnp.int32)]).reshape(4, g4e).T.reshape(-1)
    dst_r = dst_p.reshape(NW, ch, LANE)
    src_r = src_p.reshape(NW, ch, LANE)
    eftT = jnp.pad(edge_features, ((0, pad_e), (0, 0))).T
    zero_blk = jnp.zeros((n_pad // NS, d), jnp.float32)

    wp2 = edge_kernel.reshape(de, d, d).transpose(1, 0, 2).reshape(d, de * d)
    wpa = jnp.concatenate([wp2, edge_bias.reshape(d, d)], axis=1)
    kz, kr, kh = (gru_kernel[:, :d], gru_kernel[:, d:2 * d],
                  gru_kernel[:, 2 * d:])
    rkz, rkr, rkh = (gru_rkernel[:, :d], gru_rkernel[:, d:2 * d],
                     gru_rkernel[:, 2 * d:])
    zer = jnp.zeros((d, d), jnp.float32)
    wbig = jnp.concatenate([
        jnp.concatenate([kz, kr, kh, zer], axis=1),
        jnp.concatenate([rkz, rkr, zer, rkh], axis=1),
    ], axis=0)
    wbt = wbig.T
    bz = (gru_bias[0, :d] + gru_bias[1, :d]).reshape(d, 1)
    br = (gru_bias[0, d:2 * d] + gru_bias[1, d:2 * d]).reshape(d, 1)
    b0h = gru_bias[0, 2 * d:].reshape(d, 1)
    b1h = gru_bias[1, 2 * d:].reshape(d, 1)

    g4 = n_pad // 4
    bgn = g4
    for cand in range(632, 7, -8):
        if g4 % cand == 0:
            bgn = cand
            break

    sc_gather = _make_sc_gather(n_pad, d, e_pad)
    sc_scatter = _make_sc_scatter(n_pad, d, e_pad)
    tc_msg = _make_tc_msg(de, d, e_pad, 512)
    tc_gru = _make_tc_gru(n_pad, d, bgn)

    h4 = jnp.pad(h, ((0, n_pad - n), (0, 0))).reshape(g4, 4 * d)
    for _ in range(STEPS):
        nbr = sc_gather(h4.reshape(n_pad, d), dst_r)
        msg4 = tc_msg(eftT, eftT, eftT, eftT,
                      nbr.reshape(e_pad // 4, 4 * d), wpa)
        aggp = sc_scatter(msg4.reshape(e_pad, d), src_r, zero_blk)
        agg4 = aggp.reshape(NC, g4, 4 * d)
        h4 = tc_gru(agg4, h4, wbt, bz, br, b0h, b1h)
    return h4.reshape(n_pad, d)[:n]


# speedup vs baseline: 8.3134x; 1.0198x over previous
import functools

import jax
import jax.numpy as jnp
from jax import lax
from jax.experimental import pallas as pl
from jax.experimental.pallas import tpu as pltpu
from jax.experimental.pallas import tpu_sc as plsc

STEPS = 4

NC = 2
NS = 16
NW = NC * NS
LANE = 128


def _mesh():
    return plsc.VectorSubcoreMesh(core_axis_name="c", subcore_axis_name="s")


_SC_PARAMS = pltpu.CompilerParams(use_tc_tiling_on_sc=False)


def _make_sc_gather(n_nodes, d, e_pad):
    epw = e_pad // NW
    ch = epw // LANE

    rpt = n_nodes // NS

    @functools.partial(
        pl.kernel,
        out_type=jax.ShapeDtypeStruct((e_pad, d), jnp.float32),
        mesh=_mesh(),
        scratch_types=[
            pltpu.VMEM((ch, LANE), jnp.int32),
            pltpu.VMEM((epw, d), jnp.float32),
            pltpu.VMEM_SHARED((n_nodes, d), jnp.float32),
            pltpu.SemaphoreType.DMA,
        ],
        compiler_params=_SC_PARAMS,
    )
    def sc_gather(h_hbm, dstr_hbm, out_hbm, idx_v, rows_v, h_s, sem):
        cid = lax.axis_index("c")
        sid = lax.axis_index("s")
        wid = sid * NC + cid
        pltpu.sync_copy(h_hbm.at[pl.ds(sid * rpt, rpt)],
                        h_s.at[pl.ds(sid * rpt, rpt)])
        pltpu.sync_copy(dstr_hbm.at[wid], idx_v)
        plsc.subcore_barrier()
        descs = [
            pltpu.async_copy(
                h_s.at[idx_v.at[j]], rows_v.at[pl.ds(j * LANE, LANE)], sem
            )
            for j in range(ch)
        ]
        for dsc in descs:
            dsc.wait()
        pltpu.sync_copy(rows_v, out_hbm.at[pl.ds(wid * epw, epw)])

    return sc_gather


def _make_sc_scatter(n_pad, d, e_pad):
    epw = e_pad // NW
    ch = epw // LANE
    rpt = n_pad // NS

    @functools.partial(
        pl.kernel,
        out_type=jax.ShapeDtypeStruct((NC, n_pad, d), jnp.float32),
        mesh=_mesh(),
        scratch_types=[
            pltpu.VMEM((ch, LANE), jnp.int32),
            pltpu.VMEM((epw, d), jnp.float32),
            pltpu.VMEM_SHARED((n_pad, d), jnp.float32),
        ],
        compiler_params=_SC_PARAMS,
    )
    def sc_scatter(msg_hbm, srcr_hbm, zero_hbm, out_hbm, idx_v, msg_v, acc_s):
        cid = lax.axis_index("c")
        sid = lax.axis_index("s")
        wid = sid * NC + cid
        pltpu.sync_copy(zero_hbm, acc_s.at[pl.ds(sid * rpt, rpt)])
        plsc.subcore_barrier()
        pltpu.sync_copy(srcr_hbm.at[wid], idx_v)
        pltpu.sync_copy(msg_hbm.at[pl.ds(wid * epw, epw)], msg_v)
        for j in range(ch):
            pltpu.sync_copy(
                msg_v.at[pl.ds(j * LANE, LANE)],
                acc_s.at[idx_v.at[j]],
                add=True,
            )
        plsc.subcore_barrier()
        pltpu.sync_copy(
            acc_s.at[pl.ds(sid * rpt, rpt)],
            out_hbm.at[cid, pl.ds(sid * rpt, rpt)],
        )

    return sc_scatter


def _msg_body(de, d, ft0_ref, ft1_ref, ft2_ref, ft3_ref, nbr4_ref, wpa_ref,
              out_ref):
    bf = jnp.bfloat16
    ytp = nbr4_ref[...].T.astype(bf)
    wpa = wpa_ref[...]
    fts = (ft0_ref, ft1_ref, ft2_ref, ft3_ref)
    parts = []
    for a in range(4):
        yt = ytp[d * a : d * (a + 1), :]
        ft = fts[a][...]
        p = jnp.concatenate(
            [yt * ft[k : k + 1, :] for k in range(de)] + [yt], axis=0)
        mt = jnp.dot(wpa, p, preferred_element_type=jnp.float32)
        parts.append(mt)
    out_ref[...] = jnp.concatenate(parts, axis=0).T


def _make_tc_msg(de, d, e_pad, bg):
    g4 = e_pad // 4
    grid = (g4 // bg,)
    nb = g4 // bg

    def ftmap(a):
        return lambda i: (0, a * nb + i)

    return pl.pallas_call(
        functools.partial(_msg_body, de, d),
        grid=grid,
        in_specs=[
            pl.BlockSpec((de, bg), ftmap(0)),
            pl.BlockSpec((de, bg), ftmap(1)),
            pl.BlockSpec((de, bg), ftmap(2)),
            pl.BlockSpec((de, bg), ftmap(3)),
            pl.BlockSpec((bg, 4 * d), lambda i: (i, 0)),
            pl.BlockSpec((d, de * d + d), lambda i: (0, 0)),
        ],
        out_specs=pl.BlockSpec((bg, 4 * d), lambda i: (i, 0)),
        out_shape=jax.ShapeDtypeStruct((g4, 4 * d), jnp.float32),
    )


def _gru_body(d, agg4_ref, h4_ref, wbt_ref, bz_ref, br_ref, b0h_ref,
              b1h_ref, out_ref):
    at = (agg4_ref[0] + agg4_ref[1]).T
    ht = h4_ref[...].T
    wbt = wbt_ref[...]
    bz, br, b0h, b1h = bz_ref[...], br_ref[...], b0h_ref[...], b1h_ref[...]
    parts = []
    for a in range(4):
        aa = at[d * a : d * (a + 1), :]
        hh_in = ht[d * a : d * (a + 1), :]
        x = jnp.concatenate([aa, hh_in], axis=0)
        m = jnp.dot(wbt, x, preferred_element_type=jnp.float32)
        z = jax.nn.sigmoid(m[0 : d, :] + bz)
        r = jax.nn.sigmoid(m[d : 2 * d, :] + br)
        cand = jnp.tanh(m[2 * d : 3 * d, :] + b0h
                        + r * (m[3 * d : 4 * d, :] + b1h))
        parts.append(z * hh_in + (1.0 - z) * cand)
    out_ref[...] = jnp.concatenate(parts, axis=0).T


def _make_tc_gru(n_pad, d, bg):
    g4 = n_pad // 4
    grid = (g4 // bg,)
    return pl.pallas_call(
        functools.partial(_gru_body, d),
        grid=grid,
        in_specs=[
            pl.BlockSpec((NC, bg, 4 * d), lambda i: (0, i, 0)),
            pl.BlockSpec((bg, 4 * d), lambda i: (i, 0)),
            pl.BlockSpec((4 * d, 2 * d), lambda i: (0, 0)),
            pl.BlockSpec((d, 1), lambda i: (0, 0)),
            pl.BlockSpec((d, 1), lambda i: (0, 0)),
            pl.BlockSpec((d, 1), lambda i: (0, 0)),
            pl.BlockSpec((d, 1), lambda i: (0, 0)),
        ],
        out_specs=pl.BlockSpec((bg, 4 * d), lambda i: (i, 0)),
        out_shape=jax.ShapeDtypeStruct((g4, 4 * d), jnp.float32),
    )


def kernel(node_features, edge_features, pair_indices, edge_kernel,
           edge_bias, gru_kernel, gru_rkernel, gru_bias):
    n, nfc_in = node_features.shape
    e, de = edge_features.shape
    d = gru_kernel.shape[0]
    assert edge_kernel.shape == (de, d * d)
    assert n % 4 == 0 and d == 32 and de == 16

    h = node_features
    if nfc_in < d:
        h = jnp.pad(h, ((0, 0), (0, d - nfc_in)))

    quant = NW * LANE
    e_pad = ((e + quant - 1) // quant) * quant
    ch = e_pad // (NW * LANE)
    rpt = -(-(n + 1) // NS)
    rpt = ((rpt + 7) // 8) * 8
    n_pad = rpt * NS

    src = pair_indices[:, 0]
    dst = pair_indices[:, 1]
    pad_e = e_pad - e
    g4e = e_pad // 4
    dst_p = jnp.concatenate(
        [dst, jnp.zeros((pad_e,), jnp.int32)]).reshape(4, g4e).T.reshape(-1)
    src_p = jnp.concatenate(
        [src, jnp.full((pad_e,), n, jnp.int32)]).reshape(4, g4e).T.reshape(-1)
    dst_r = dst_p.reshape(NW, ch, LANE)
    src_r = src_p.reshape(NW, ch, LANE)
    eftT = jnp.pad(edge_features, ((0, pad_e), (0, 0))).T.astype(
        jnp.bfloat16)
    zero_blk = jnp.zeros((n_pad // NS, d), jnp.float32)

    wp2 = edge_kernel.reshape(de, d, d).transpose(1, 0, 2).reshape(d, de * d)
    wpa = jnp.concatenate(
        [wp2, edge_bias.reshape(d, d)], axis=1).astype(jnp.bfloat16)
    kz, kr, kh = (gru_kernel[:, :d], gru_kernel[:, d:2 * d],
                  gru_kernel[:, 2 * d:])
    rkz, rkr, rkh = (gru_rkernel[:, :d], gru_rkernel[:, d:2 * d],
                     gru_rkernel[:, 2 * d:])
    zer = jnp.zeros((d, d), jnp.float32)
    wbig = jnp.concatenate([
        jnp.concatenate([kz, kr, kh, zer], axis=1),
        jnp.concatenate([rkz, rkr, zer, rkh], axis=1),
    ], axis=0)
    wbt = wbig.T
    bz = (gru_bias[0, :d] + gru_bias[1, :d]).reshape(d, 1)
    br = (gru_bias[0, d:2 * d] + gru_bias[1, d:2 * d]).reshape(d, 1)
    b0h = gru_bias[0, 2 * d:].reshape(d, 1)
    b1h = gru_bias[1, 2 * d:].reshape(d, 1)

    g4 = n_pad // 4
    bgn = g4
    for cand in range(632, 7, -8):
        if g4 % cand == 0:
            bgn = cand
            break

    sc_gather = _make_sc_gather(n_pad, d, e_pad)
    sc_scatter = _make_sc_scatter(n_pad, d, e_pad)
    tc_msg = _make_tc_msg(de, d, e_pad, 512)
    tc_gru = _make_tc_gru(n_pad, d, bgn)

    h4 = jnp.pad(h, ((0, n_pad - n), (0, 0))).reshape(g4, 4 * d)
    for _ in range(STEPS):
        nbr = sc_gather(h4.reshape(n_pad, d), dst_r)
        msg4 = tc_msg(eftT, eftT, eftT, eftT,
                      nbr.reshape(e_pad // 4, 4 * d), wpa)
        aggp = sc_scatter(msg4.reshape(e_pad, d), src_r, zero_blk)
        agg4 = aggp.reshape(NC, g4, 4 * d)
        h4 = tc_gru(agg4, h4, wbt, bz, br, b0h, b1h)
    return h4.reshape(n_pad, d)[:n]


# speedup vs baseline: 10.0909x; 1.2138x over previous
import functools

import jax
import jax.numpy as jnp
from jax import lax
from jax.experimental import pallas as pl
from jax.experimental.pallas import tpu as pltpu
from jax.experimental.pallas import tpu_sc as plsc

STEPS = 4

NC = 2
NS = 16
NW = NC * NS
LANE = 128


def _mesh():
    return plsc.VectorSubcoreMesh(core_axis_name="c", subcore_axis_name="s")


_SC_PARAMS = pltpu.CompilerParams(use_tc_tiling_on_sc=False)


def _make_sc_gather(n_nodes, d, e_pad):
    epw = e_pad // NW
    ch = epw // LANE

    rpt = n_nodes // NS

    @functools.partial(
        pl.kernel,
        out_type=jax.ShapeDtypeStruct((e_pad, d), jnp.float32),
        mesh=_mesh(),
        scratch_types=[
            pltpu.VMEM((ch, LANE), jnp.int32),
            pltpu.VMEM((epw, d), jnp.float32),
            pltpu.VMEM_SHARED((n_nodes, d), jnp.float32),
            pltpu.SemaphoreType.DMA,
        ],
        compiler_params=_SC_PARAMS,
    )
    def sc_gather(h_hbm, dstr_hbm, out_hbm, idx_v, rows_v, h_s, sem):
        cid = lax.axis_index("c")
        sid = lax.axis_index("s")
        wid = sid * NC + cid
        pltpu.sync_copy(h_hbm.at[pl.ds(sid * rpt, rpt)],
                        h_s.at[pl.ds(sid * rpt, rpt)])
        pltpu.sync_copy(dstr_hbm.at[wid], idx_v)
        plsc.subcore_barrier()
        descs = [
            pltpu.async_copy(
                h_s.at[idx_v.at[j]], rows_v.at[pl.ds(j * LANE, LANE)], sem
            )
            for j in range(ch)
        ]
        for dsc in descs:
            dsc.wait()
        pltpu.sync_copy(rows_v, out_hbm.at[pl.ds(wid * epw, epw)])

    return sc_gather


def _make_sc_scatter(n_pad, d, e_pad):
    epw = e_pad // NW
    ch = epw // LANE
    rpt = n_pad // NS

    @functools.partial(
        pl.kernel,
        out_type=jax.ShapeDtypeStruct((NC, n_pad, d), jnp.float32),
        mesh=_mesh(),
        scratch_types=[
            pltpu.VMEM((ch, LANE), jnp.int32),
            pltpu.VMEM((epw, d), jnp.float32),
            pltpu.VMEM_SHARED((n_pad, d), jnp.float32),
        ],
        compiler_params=_SC_PARAMS,
    )
    def sc_scatter(msg_hbm, srcr_hbm, zero_hbm, out_hbm, idx_v, msg_v, acc_s):
        cid = lax.axis_index("c")
        sid = lax.axis_index("s")
        wid = sid * NC + cid
        pltpu.sync_copy(zero_hbm, acc_s.at[pl.ds(sid * rpt, rpt)])
        plsc.subcore_barrier()
        pltpu.sync_copy(srcr_hbm.at[wid], idx_v)
        pltpu.sync_copy(msg_hbm.at[pl.ds(wid * epw, epw)], msg_v)
        for j in range(ch):
            pltpu.sync_copy(
                msg_v.at[pl.ds(j * LANE, LANE)],
                acc_s.at[idx_v.at[j]],
                add=True,
            )
        plsc.subcore_barrier()
        pltpu.sync_copy(
            acc_s.at[pl.ds(sid * rpt, rpt)],
            out_hbm.at[cid, pl.ds(sid * rpt, rpt)],
        )

    return sc_scatter


def _msg_body(de, d, ft0_ref, ft1_ref, ft2_ref, ft3_ref, nbr4_ref, wpa_ref,
              out_ref):
    bf = jnp.bfloat16
    ytp = nbr4_ref[...].T.astype(bf)
    wpa = wpa_ref[...]
    fts = (ft0_ref, ft1_ref, ft2_ref, ft3_ref)
    parts = []
    for a in range(4):
        yt = ytp[d * a : d * (a + 1), :]
        ft = fts[a][...]
        p = jnp.concatenate(
            [yt * ft[k : k + 1, :] for k in range(de)] + [yt], axis=0)
        mt = jnp.dot(wpa, p, preferred_element_type=jnp.float32)
        parts.append(mt)
    out_ref[...] = jnp.concatenate(parts, axis=0).T


def _make_tc_msg(de, d, e_pad, bg):
    g4 = e_pad // 4
    grid = (g4 // bg,)
    nb = g4 // bg

    def ftmap(a):
        return lambda i: (0, a * nb + i)

    return pl.pallas_call(
        functools.partial(_msg_body, de, d),
        grid=grid,
        in_specs=[
            pl.BlockSpec((de, bg), ftmap(0)),
            pl.BlockSpec((de, bg), ftmap(1)),
            pl.BlockSpec((de, bg), ftmap(2)),
            pl.BlockSpec((de, bg), ftmap(3)),
            pl.BlockSpec((bg, 4 * d), lambda i: (i, 0)),
            pl.BlockSpec((d, de * d + d), lambda i: (0, 0)),
        ],
        out_specs=pl.BlockSpec((bg, 4 * d), lambda i: (i, 0)),
        out_shape=jax.ShapeDtypeStruct((g4, 4 * d), jnp.float32),
    )


def _gru_body(d, agg4_ref, h4_ref, wbt_ref, bz_ref, br_ref, b0h_ref,
              b1h_ref, out_ref):
    at = (agg4_ref[0] + agg4_ref[1]).T
    ht = h4_ref[...].T
    wbt = wbt_ref[...]
    bz, br, b0h, b1h = bz_ref[...], br_ref[...], b0h_ref[...], b1h_ref[...]
    parts = []
    for a in range(4):
        aa = at[d * a : d * (a + 1), :]
        hh_in = ht[d * a : d * (a + 1), :]
        x = jnp.concatenate([aa, hh_in], axis=0)
        m = jnp.dot(wbt, x, preferred_element_type=jnp.float32)
        z = jax.nn.sigmoid(m[0 : d, :] + bz)
        r = jax.nn.sigmoid(m[d : 2 * d, :] + br)
        cand = jnp.tanh(m[2 * d : 3 * d, :] + b0h
                        + r * (m[3 * d : 4 * d, :] + b1h))
        parts.append(z * hh_in + (1.0 - z) * cand)
    out_ref[...] = jnp.concatenate(parts, axis=0).T


def _make_tc_gru(n_pad, d, bg):
    g4 = n_pad // 4
    grid = (g4 // bg,)
    return pl.pallas_call(
        functools.partial(_gru_body, d),
        grid=grid,
        in_specs=[
            pl.BlockSpec((NC, bg, 4 * d), lambda i: (0, i, 0)),
            pl.BlockSpec((bg, 4 * d), lambda i: (i, 0)),
            pl.BlockSpec((4 * d, 2 * d), lambda i: (0, 0)),
            pl.BlockSpec((d, 1), lambda i: (0, 0)),
            pl.BlockSpec((d, 1), lambda i: (0, 0)),
            pl.BlockSpec((d, 1), lambda i: (0, 0)),
            pl.BlockSpec((d, 1), lambda i: (0, 0)),
        ],
        out_specs=pl.BlockSpec((bg, 4 * d), lambda i: (i, 0)),
        out_shape=jax.ShapeDtypeStruct((g4, 4 * d), jnp.float32),
    )


def kernel(node_features, edge_features, pair_indices, edge_kernel,
           edge_bias, gru_kernel, gru_rkernel, gru_bias):
    n, nfc_in = node_features.shape
    e, de = edge_features.shape
    d = gru_kernel.shape[0]
    assert edge_kernel.shape == (de, d * d)
    assert n % 4 == 0 and d == 32 and de == 16

    h = node_features
    if nfc_in < d:
        h = jnp.pad(h, ((0, 0), (0, d - nfc_in)))

    quant = NW * LANE
    e_pad = ((e + quant - 1) // quant) * quant
    ch = e_pad // (NW * LANE)
    rpt = -(-(n + 1) // NS)
    rpt = ((rpt + 7) // 8) * 8
    n_pad = rpt * NS

    src = pair_indices[:, 0]
    dst = pair_indices[:, 1]
    pad_e = e_pad - e
    g4e = e_pad // 4
    dst_p = jnp.concatenate(
        [dst, jnp.zeros((pad_e,), jnp.int32)]).reshape(4, g4e).T.reshape(-1)
    src_p = jnp.concatenate(
        [src, jnp.full((pad_e,), n, jnp.int32)]).reshape(4, g4e).T.reshape(-1)
    dst_r = dst_p.reshape(NW, ch, LANE)
    src_r = src_p.reshape(NW, ch, LANE)
    eftT = jnp.pad(edge_features, ((0, pad_e), (0, 0))).T.astype(
        jnp.bfloat16)
    zero_blk = jnp.zeros((n_pad // NS, d), jnp.float32)

    wp2 = edge_kernel.reshape(de, d, d).transpose(1, 0, 2).reshape(d, de * d)
    wpa = jnp.concatenate(
        [wp2, edge_bias.reshape(d, d)], axis=1).astype(jnp.bfloat16)
    kz, kr, kh = (gru_kernel[:, :d], gru_kernel[:, d:2 * d],
                  gru_kernel[:, 2 * d:])
    rkz, rkr, rkh = (gru_rkernel[:, :d], gru_rkernel[:, d:2 * d],
                     gru_rkernel[:, 2 * d:])
    zer = jnp.zeros((d, d), jnp.float32)
    wbig = jnp.concatenate([
        jnp.concatenate([kz, kr, kh, zer], axis=1),
        jnp.concatenate([rkz, rkr, zer, rkh], axis=1),
    ], axis=0)
    wbt = wbig.T
    bz = (gru_bias[0, :d] + gru_bias[1, :d]).reshape(d, 1)
    br = (gru_bias[0, d:2 * d] + gru_bias[1, d:2 * d]).reshape(d, 1)
    b0h = gru_bias[0, 2 * d:].reshape(d, 1)
    b1h = gru_bias[1, 2 * d:].reshape(d, 1)

    g4 = n_pad // 4
    bgn = g4
    for cand in range(632, 7, -8):
        if g4 % cand == 0:
            bgn = cand
            break

    sc_gather = _make_sc_gather(n_pad, d, e_pad)
    sc_scatter = _make_sc_scatter(n_pad, d, e_pad)
    tc_msg = _make_tc_msg(de, d, e_pad, 2048)
    tc_gru = _make_tc_gru(n_pad, d, bgn)

    h4 = jnp.pad(h, ((0, n_pad - n), (0, 0))).reshape(g4, 4 * d)
    for _ in range(STEPS):
        nbr = sc_gather(h4.reshape(n_pad, d), dst_r)
        msg4 = tc_msg(eftT, eftT, eftT, eftT,
                      nbr.reshape(e_pad // 4, 4 * d), wpa)
        aggp = sc_scatter(msg4.reshape(e_pad, d), src_r, zero_blk)
        agg4 = aggp.reshape(NC, g4, 4 * d)
        h4 = tc_gru(agg4, h4, wbt, bz, br, b0h, b1h)
    return h4.reshape(n_pad, d)[:n]


# speedup vs baseline: 10.2351x; 1.0143x over previous
import functools

import jax
import jax.numpy as jnp
from jax import lax
from jax.experimental import pallas as pl
from jax.experimental.pallas import tpu as pltpu
from jax.experimental.pallas import tpu_sc as plsc

STEPS = 4

NC = 2
NS = 16
NW = NC * NS
LANE = 128


def _mesh():
    return plsc.VectorSubcoreMesh(core_axis_name="c", subcore_axis_name="s")


_SC_PARAMS = pltpu.CompilerParams(use_tc_tiling_on_sc=False)


def _make_sc_gather(n_nodes, d, e_pad):
    epw = e_pad // NW
    ch = epw // LANE

    rpt = n_nodes // NS

    @functools.partial(
        pl.kernel,
        out_type=jax.ShapeDtypeStruct((e_pad, d), jnp.float32),
        mesh=_mesh(),
        scratch_types=[
            pltpu.VMEM((ch, LANE), jnp.int32),
            pltpu.VMEM((epw, d), jnp.float32),
            pltpu.VMEM_SHARED((n_nodes, d), jnp.float32),
            pltpu.SemaphoreType.DMA,
        ],
        compiler_params=_SC_PARAMS,
    )
    def sc_gather(h_hbm, dstr_hbm, out_hbm, idx_v, rows_v, h_s, sem):
        cid = lax.axis_index("c")
        sid = lax.axis_index("s")
        wid = sid * NC + cid
        pltpu.sync_copy(h_hbm.at[pl.ds(sid * rpt, rpt)],
                        h_s.at[pl.ds(sid * rpt, rpt)])
        pltpu.sync_copy(dstr_hbm.at[wid], idx_v)
        plsc.subcore_barrier()
        descs = [
            pltpu.async_copy(
                h_s.at[idx_v.at[j]], rows_v.at[pl.ds(j * LANE, LANE)], sem
            )
            for j in range(ch)
        ]
        for dsc in descs:
            dsc.wait()
        pltpu.sync_copy(rows_v, out_hbm.at[pl.ds(wid * epw, epw)])

    return sc_gather


def _make_sc_scatter(n_pad, d, e_pad):
    epw = e_pad // NW
    ch = epw // LANE
    rpt = n_pad // NS

    @functools.partial(
        pl.kernel,
        out_type=jax.ShapeDtypeStruct((NC, n_pad, d), jnp.float32),
        mesh=_mesh(),
        scratch_types=[
            pltpu.VMEM((ch, LANE), jnp.int32),
            pltpu.VMEM((epw, d), jnp.float32),
            pltpu.VMEM_SHARED((n_pad, d), jnp.float32),
        ],
        compiler_params=_SC_PARAMS,
    )
    def sc_scatter(msg_hbm, srcr_hbm, zero_hbm, out_hbm, idx_v, msg_v, acc_s):
        cid = lax.axis_index("c")
        sid = lax.axis_index("s")
        wid = sid * NC + cid
        pltpu.sync_copy(zero_hbm, acc_s.at[pl.ds(sid * rpt, rpt)])
        plsc.subcore_barrier()
        pltpu.sync_copy(srcr_hbm.at[wid], idx_v)
        pltpu.sync_copy(msg_hbm.at[pl.ds(wid * epw, epw)], msg_v)
        for j in range(ch):
            pltpu.sync_copy(
                msg_v.at[pl.ds(j * LANE, LANE)],
                acc_s.at[idx_v.at[j]],
                add=True,
            )
        plsc.subcore_barrier()
        pltpu.sync_copy(
            acc_s.at[pl.ds(sid * rpt, rpt)],
            out_hbm.at[cid, pl.ds(sid * rpt, rpt)],
        )

    return sc_scatter


def _msg_body(de, d, ft0_ref, ft1_ref, ft2_ref, ft3_ref, nbr4_ref, wpa_ref,
              out_ref):
    bf = jnp.bfloat16
    ytp = nbr4_ref[...].T.astype(bf)
    wpa = wpa_ref[...]
    fts = (ft0_ref, ft1_ref, ft2_ref, ft3_ref)
    parts = []
    for a in range(4):
        yt = ytp[d * a : d * (a + 1), :]
        ft = fts[a][...]
        p = jnp.concatenate(
            [yt * ft[k : k + 1, :] for k in range(de)] + [yt], axis=0)
        mt = jnp.dot(wpa, p, preferred_element_type=jnp.float32)
        parts.append(mt)
    out_ref[...] = jnp.concatenate(parts, axis=0).T


def _make_tc_msg(de, d, e_pad, bg):
    g4 = e_pad // 4
    grid = (g4 // bg,)
    nb = g4 // bg

    def ftmap(a):
        return lambda i: (0, a * nb + i)

    return pl.pallas_call(
        functools.partial(_msg_body, de, d),
        grid=grid,
        in_specs=[
            pl.BlockSpec((de, bg), ftmap(0)),
            pl.BlockSpec((de, bg), ftmap(1)),
            pl.BlockSpec((de, bg), ftmap(2)),
            pl.BlockSpec((de, bg), ftmap(3)),
            pl.BlockSpec((bg, 4 * d), lambda i: (i, 0)),
            pl.BlockSpec((d, de * d + d), lambda i: (0, 0)),
        ],
        out_specs=pl.BlockSpec((bg, 4 * d), lambda i: (i, 0)),
        out_shape=jax.ShapeDtypeStruct((g4, 4 * d), jnp.float32),
    )


def _gru_body(d, agg4_ref, h4_ref, wbt_ref, bz_ref, br_ref, b0h_ref,
              b1h_ref, out_ref):
    at = (agg4_ref[0] + agg4_ref[1]).T
    ht = h4_ref[...].T
    wbt = wbt_ref[...]
    bz, br, b0h, b1h = bz_ref[...], br_ref[...], b0h_ref[...], b1h_ref[...]
    parts = []
    for a in range(4):
        aa = at[d * a : d * (a + 1), :]
        hh_in = ht[d * a : d * (a + 1), :]
        x = jnp.concatenate([aa, hh_in], axis=0)
        m = jnp.dot(wbt, x, preferred_element_type=jnp.float32)
        z = jax.nn.sigmoid(m[0 : d, :] + bz)
        r = jax.nn.sigmoid(m[d : 2 * d, :] + br)
        cand = jnp.tanh(m[2 * d : 3 * d, :] + b0h
                        + r * (m[3 * d : 4 * d, :] + b1h))
        parts.append(z * hh_in + (1.0 - z) * cand)
    out_ref[...] = jnp.concatenate(parts, axis=0).T


def _make_tc_gru(n_pad, d, bg):
    g4 = n_pad // 4
    grid = (g4 // bg,)
    return pl.pallas_call(
        functools.partial(_gru_body, d),
        grid=grid,
        in_specs=[
            pl.BlockSpec((NC, bg, 4 * d), lambda i: (0, i, 0)),
            pl.BlockSpec((bg, 4 * d), lambda i: (i, 0)),
            pl.BlockSpec((4 * d, 2 * d), lambda i: (0, 0)),
            pl.BlockSpec((d, 1), lambda i: (0, 0)),
            pl.BlockSpec((d, 1), lambda i: (0, 0)),
            pl.BlockSpec((d, 1), lambda i: (0, 0)),
            pl.BlockSpec((d, 1), lambda i: (0, 0)),
        ],
        out_specs=pl.BlockSpec((bg, 4 * d), lambda i: (i, 0)),
        out_shape=jax.ShapeDtypeStruct((g4, 4 * d), jnp.float32),
    )


def kernel(node_features, edge_features, pair_indices, edge_kernel,
           edge_bias, gru_kernel, gru_rkernel, gru_bias):
    n, nfc_in = node_features.shape
    e, de = edge_features.shape
    d = gru_kernel.shape[0]
    assert edge_kernel.shape == (de, d * d)
    assert n % 4 == 0 and d == 32 and de == 16

    h = node_features
    if nfc_in < d:
        h = jnp.pad(h, ((0, 0), (0, d - nfc_in)))

    quant = NW * LANE
    e_pad = ((e + quant - 1) // quant) * quant
    ch = e_pad // (NW * LANE)
    rpt = -(-(n + 1) // NS)
    rpt = ((rpt + 7) // 8) * 8
    n_pad = rpt * NS

    src = pair_indices[:, 0]
    dst = pair_indices[:, 1]
    pad_e = e_pad - e
    g4e = e_pad // 4
    dst_p = jnp.concatenate(
        [dst, jnp.zeros((pad_e,), jnp.int32)]).reshape(4, g4e).T.reshape(-1)
    src_p = jnp.concatenate(
        [src, jnp.full((pad_e,), n, jnp.int32)]).reshape(4, g4e).T.reshape(-1)
    dst_r = dst_p.reshape(NW, ch, LANE)
    src_r = src_p.reshape(NW, ch, LANE)
    eftT = jnp.pad(edge_features, ((0, pad_e), (0, 0))).T.astype(
        jnp.bfloat16)
    zero_blk = jnp.zeros((n_pad // NS, d), jnp.float32)

    wp2 = edge_kernel.reshape(de, d, d).transpose(1, 0, 2).reshape(d, de * d)
    wpa = jnp.concatenate(
        [wp2, edge_bias.reshape(d, d)], axis=1).astype(jnp.bfloat16)
    kz, kr, kh = (gru_kernel[:, :d], gru_kernel[:, d:2 * d],
                  gru_kernel[:, 2 * d:])
    rkz, rkr, rkh = (gru_rkernel[:, :d], gru_rkernel[:, d:2 * d],
                     gru_rkernel[:, 2 * d:])
    zer = jnp.zeros((d, d), jnp.float32)
    wbig = jnp.concatenate([
        jnp.concatenate([kz, kr, kh, zer], axis=1),
        jnp.concatenate([rkz, rkr, zer, rkh], axis=1),
    ], axis=0)
    wbt = wbig.T
    bz = (gru_bias[0, :d] + gru_bias[1, :d]).reshape(d, 1)
    br = (gru_bias[0, d:2 * d] + gru_bias[1, d:2 * d]).reshape(d, 1)
    b0h = gru_bias[0, 2 * d:].reshape(d, 1)
    b1h = gru_bias[1, 2 * d:].reshape(d, 1)

    g4 = n_pad // 4
    bgn = g4
    for cand in range(632, 7, -8):
        if g4 % cand == 0:
            bgn = cand
            break

    sc_gather = _make_sc_gather(n_pad, d, e_pad)
    sc_scatter = _make_sc_scatter(n_pad, d, e_pad)
    tc_msg = _make_tc_msg(de, d, e_pad, 4096)
    tc_gru = _make_tc_gru(n_pad, d, bgn)

    h4 = jnp.pad(h, ((0, n_pad - n), (0, 0))).reshape(g4, 4 * d)
    for _ in range(STEPS):
        nbr = sc_gather(h4.reshape(n_pad, d), dst_r)
        msg4 = tc_msg(eftT, eftT, eftT, eftT,
                      nbr.reshape(e_pad // 4, 4 * d), wpa)
        aggp = sc_scatter(msg4.reshape(e_pad, d), src_r, zero_blk)
        agg4 = aggp.reshape(NC, g4, 4 * d)
        h4 = tc_gru(agg4, h4, wbt, bz, br, b0h, b1h)
    return h4.reshape(n_pad, d)[:n]
